# baseline (speedup 1.0000x reference)
import functools

import jax
import jax.numpy as jnp
from jax import lax
from jax.experimental import pallas as pl
from jax.experimental.pallas import tpu as pltpu
from jax.experimental.pallas import tpu_sc as plsc

N_IN = 768
N_LAT = 16384
TOPK = 64
ROWS = 4096

B1, L1 = 512, 2048
B3, L3 = 512, 2048

ROWS_H = ROWS // 2
NWORK = 32
RPW = ROWS_H // NWORK
NVREG = N_LAT // 16
NGRP = 8

INT_MIN = -0x80000000


def _enc_body(x_ref, b_ref, w_ref, benc_ref, out_ref):
    xc = x_ref[...] - b_ref[...]
    out_ref[...] = jax.lax.dot_general(
        xc, w_ref[...], (((1,), (1,)), ((), ())),
        preferred_element_type=jnp.float32,
    ) + benc_ref[...]


def _mono(x):
    u = jax.lax.bitcast_convert_type(x, jnp.int32)
    return jnp.where(u >= 0, u, u ^ jnp.int32(0x7FFFFFFF))


def _key_to_float(k):
    bits = jnp.where(k >= 0, k, k ^ jnp.int32(0x7FFFFFFF))
    return jax.lax.bitcast_convert_type(bits, jnp.float32)


def _sc_body(lg_hbm, out_hbm, rows_v, cand_v, th_v, sem):
    wid = lax.axis_index("s") * 2 + lax.axis_index("c")
    base_row = wid * RPW
    lanes = lax.iota(jnp.int32, 16)
    lane0 = lanes == 0
    lane_base = lanes * NVREG

    def spl_i(x):
        return jnp.full((16,), x, jnp.int32)

    kvec = spl_i(TOPK)
    zero = jnp.zeros((16,), jnp.int32)
    tmin = jnp.full((16,), INT_MIN, jnp.int32)
    onev = jnp.full((16,), 1, jnp.int32)

    def start_copy(rl):
        return pltpu.async_copy(
            lg_hbm.at[base_row + rl],
            rows_v.at[pl.ds((rl % 2) * N_LAT, N_LAT)], sem)

    start_copy(0).wait()

    def row_body(rl, carry):
        @pl.when(rl + 1 < RPW)
        def _prefetch():
            start_copy(rl + 1)

        @pl.when(rl >= 1)
        def _wait_mine():
            pltpu.make_async_copy(
                lg_hbm.at[base_row + rl],
                rows_v.at[pl.ds((rl % 2) * N_LAT, N_LAT)], sem).wait()

        roff = (rl % 2) * N_LAT

        def p1(j, accs):
            a = list(accs)
            for u in range(2):
                for g in range(NGRP):
                    idx = ((j * 2 + u) * NGRP + g) * 16
                    a[g] = jnp.maximum(a[g], rows_v[pl.ds(roff + idx, 16)])
            return tuple(a)
        accs = lax.fori_loop(
            0, NVREG // (2 * NGRP), p1,
            tuple(jnp.full((16,), -jnp.inf, jnp.float32) for _ in range(NGRP)),
        )
        mks = [_mono(a) for a in accs]

        def cnt_ge_groups(cand):
            c = jnp.where(mks[0] >= cand, onev, zero)
            for mk in mks[1:]:
                c = c + jnp.where(mk >= cand, onev, zero)
            return spl_i(jnp.sum(c))

        c0 = cnt_ge_groups(zero)
        t = jnp.where(c0 >= kvec, zero, tmin)

        def p2(i, t):
            cand = t | jnp.left_shift(onev, spl_i(jnp.int32(30) - i))
            c = cnt_ge_groups(cand)
            return jnp.where(c >= kvec, cand, t)
        t0 = lax.fori_loop(0, 31, p2, t)
        t0f = _key_to_float(t0)

        def p3(jj, cnt_l):
            vs, ms, offs = [], [], []
            c = cnt_l
            for g in range(8):
                v = rows_v[pl.ds(roff + (jj * 8 + g) * 16, 16)]
                m = v >= t0f
                vs.append(v)
                ms.append(m)
                offs.append(c)
                c = c + jnp.where(m, onev, zero)
            for g in range(8):
                plsc.store_scatter(cand_v, [lane_base + offs[g]], vs[g],
                                   mask=ms[g])
            return c
        cnt_l = lax.fori_loop(0, NVREG // 8, p3, zero)
        nv = jnp.max(cnt_l)
        nv4 = (nv + 3) // 4

        def cnt_ge_cands(candf):
            def inner(j4, acc):
                for u in range(4):
                    jj = j4 * 4 + u
                    cv = plsc.load_gather(cand_v, [lane_base + jj])
                    ok = (spl_i(jj) < cnt_l) & (cv >= candf)
                    acc = acc + jnp.where(ok, onev, zero)
                return acc
            accv = lax.fori_loop(0, nv4, inner, zero)
            return spl_i(jnp.sum(accv))

        c0e = cnt_ge_cands(jnp.zeros((16,), jnp.float32))
        te = jnp.where(c0e >= kvec, zero, tmin)

        def p4(i, t):
            cand = t | jnp.left_shift(onev, spl_i(jnp.int32(30) - i))
            c = cnt_ge_cands(_key_to_float(cand))
            return jnp.where(c >= kvec, cand, t)
        t1 = lax.fori_loop(0, 31, p4, te)

        plsc.store_scatter(th_v, [spl_i(rl)], _key_to_float(t1), mask=lane0)
        return carry

    lax.fori_loop(0, RPW, row_body, jnp.int32(0))
    pltpu.sync_copy(th_v, out_hbm.at[pl.ds(base_row, RPW)])


def _sc_thresholds(logits2d):
    mesh = plsc.VectorSubcoreMesh(core_axis_name="c", subcore_axis_name="s")
    return pl.kernel(
        _sc_body,
        mesh=mesh,
        out_type=jax.ShapeDtypeStruct((ROWS_H,), jnp.float32),
        scratch_types=[
            pltpu.VMEM((2 * N_LAT,), jnp.float32),
            pltpu.VMEM((N_LAT + 64,), jnp.float32),
            pltpu.VMEM((RPW,), jnp.float32),
            pltpu.SemaphoreType.DMA,
        ],
        compiler_params=pltpu.CompilerParams(needs_layout_passes=False),
    )(logits2d)


def _adec_body(lg_ref, th_ref, wd_ref, b_ref, alpha_ref, mask_ref, xhat_ref):
    k = pl.program_id(1)
    v = lg_ref[...]
    sel = v >= th_ref[...]
    alpha = jnp.where(sel, v, 0.0)
    alpha_ref[...] = alpha
    mask_ref[...] = sel & (v != 0.0)

    @pl.when(k == 0)
    def _init():
        xhat_ref[...] = jnp.broadcast_to(b_ref[...], (B3, N_IN))

    xhat_ref[...] += jax.lax.dot_general(
        alpha, wd_ref[...], (((1,), (1,)), ((), ())),
        preferred_element_type=jnp.float32,
    )


def _encode_half(xh, b2, W_enc, benc2):
    return pl.pallas_call(
        _enc_body,
        grid=(N_LAT // L1, ROWS_H // B1),
        in_specs=[
            pl.BlockSpec((B1, N_IN), lambda l, r: (r, 0)),
            pl.BlockSpec((1, N_IN), lambda l, r: (0, 0)),
            pl.BlockSpec((L1, N_IN), lambda l, r: (l, 0)),
            pl.BlockSpec((1, L1), lambda l, r: (0, l)),
        ],
        out_specs=pl.BlockSpec((B1, L1), lambda l, r: (r, l)),
        out_shape=jax.ShapeDtypeStruct((ROWS_H, N_LAT), jnp.float32),
        compiler_params=pltpu.CompilerParams(
            dimension_semantics=("arbitrary", "arbitrary"),
        ),
    )(xh, b2, W_enc, benc2)


def _apply_decode_half(logits_h, thresh_h, W_dec, b2, half, carry):
    rb = (half * ROWS_H) // B3

    in_specs = [
        pl.BlockSpec((B3, L3), lambda r, k: (r, k)),
        pl.BlockSpec((B3, 1), lambda r, k: (r, 0)),
        pl.BlockSpec((N_IN, L3), lambda r, k: (0, k)),
        pl.BlockSpec((1, N_IN), lambda r, k: (0, 0)),
    ]
    args = [logits_h, thresh_h, W_dec, b2]
    aliases = {}
    if carry is not None:
        in_specs += [pl.BlockSpec(memory_space=pl.ANY)] * 3
        args += list(carry)
        aliases = {4: 0, 5: 1, 6: 2}

    def body(lg_ref, th_ref, wd_ref, b_ref, *rest):
        alpha_ref, mask_ref, xhat_ref = rest[-3:]
        _adec_body(lg_ref, th_ref, wd_ref, b_ref,
                   alpha_ref, mask_ref, xhat_ref)

    return pl.pallas_call(
        body,
        grid=(ROWS_H // B3, N_LAT // L3),
        in_specs=in_specs,
        out_specs=[
            pl.BlockSpec((B3, L3), lambda r, k: (r + rb, k)),
            pl.BlockSpec((B3, L3), lambda r, k: (r + rb, k)),
            pl.BlockSpec((B3, N_IN), lambda r, k: (r + rb, 0)),
        ],
        out_shape=[
            jax.ShapeDtypeStruct((ROWS, N_LAT), jnp.float32),
            jax.ShapeDtypeStruct((ROWS, N_LAT), jnp.bool_),
            jax.ShapeDtypeStruct((ROWS, N_IN), jnp.float32),
        ],
        input_output_aliases=aliases,
        compiler_params=pltpu.CompilerParams(
            dimension_semantics=("arbitrary", "arbitrary"),
        ),
    )(*args)


@jax.jit
def kernel(x, b, W_enc, b_enc, W_dec, miss_counts):
    del miss_counts
    b2 = b.reshape(1, N_IN)
    benc2 = b_enc.reshape(1, N_LAT)

    lg0 = _encode_half(x[:ROWS_H], b2, W_enc, benc2)
    th0 = _sc_thresholds(lg0).reshape(ROWS_H, 1)
    lg1 = _encode_half(x[ROWS_H:], b2, W_enc, benc2)
    th1 = _sc_thresholds(lg1).reshape(ROWS_H, 1)

    carry = _apply_decode_half(lg0, th0, W_dec, b2, 0, None)
    alpha, mask, xhat = _apply_decode_half(lg1, th1, W_dec, b2, 1, carry)
    return (xhat, alpha, mask)


# speedup vs baseline: 1.0834x; 1.0834x over previous
import functools

import jax
import jax.numpy as jnp
from jax import lax
from jax.experimental import pallas as pl
from jax.experimental.pallas import tpu as pltpu
from jax.experimental.pallas import tpu_sc as plsc

N_IN = 768
N_LAT = 16384
TOPK = 64
ROWS = 4096

B1, L1 = 512, 2048
B3, L3 = 512, 2048

NWORK = 32
RPW = ROWS // NWORK
NVREG = N_LAT // 16
NGRP = 8

INT_MIN = -0x80000000


def _enc_body(x_ref, b_ref, w_ref, benc_ref, out_ref):
    xc = x_ref[...] - b_ref[...]
    out_ref[...] = jax.lax.dot_general(
        xc, w_ref[...], (((1,), (1,)), ((), ())),
        preferred_element_type=jnp.float32,
    ) + benc_ref[...]


def _mono(x):
    u = jax.lax.bitcast_convert_type(x, jnp.int32)
    return jnp.where(u >= 0, u, u ^ jnp.int32(0x7FFFFFFF))


def _key_to_float(k):
    bits = jnp.where(k >= 0, k, k ^ jnp.int32(0x7FFFFFFF))
    return jax.lax.bitcast_convert_type(bits, jnp.float32)


def _sc_body(lg_hbm, out_hbm, rows_v, cand_v, th_v, sem):
    wid = lax.axis_index("s") * 2 + lax.axis_index("c")
    base_row = wid * RPW
    lanes = lax.iota(jnp.int32, 16)
    lane0 = lanes == 0
    lane_base = lanes * NVREG

    def spl_i(x):
        return jnp.full((16,), x, jnp.int32)

    kvec = spl_i(TOPK)
    zero = jnp.zeros((16,), jnp.int32)
    tmin = jnp.full((16,), INT_MIN, jnp.int32)
    onev = jnp.full((16,), 1, jnp.int32)

    def start_copy(rl):
        return pltpu.async_copy(
            lg_hbm.at[base_row + rl],
            rows_v.at[pl.ds((rl % 2) * N_LAT, N_LAT)], sem)

    start_copy(0).wait()

    def row_body(rl, carry):
        @pl.when(rl + 1 < RPW)
        def _prefetch():
            start_copy(rl + 1)

        @pl.when(rl >= 1)
        def _wait_mine():
            pltpu.make_async_copy(
                lg_hbm.at[base_row + rl],
                rows_v.at[pl.ds((rl % 2) * N_LAT, N_LAT)], sem).wait()

        roff = (rl % 2) * N_LAT

        def p1(j, accs):
            a = list(accs)
            for u in range(2):
                for g in range(NGRP):
                    idx = ((j * 2 + u) * NGRP + g) * 16
                    a[g] = jnp.maximum(a[g], rows_v[pl.ds(roff + idx, 16)])
            return tuple(a)
        accs = lax.fori_loop(
            0, NVREG // (2 * NGRP), p1,
            tuple(jnp.full((16,), -jnp.inf, jnp.float32) for _ in range(NGRP)),
        )
        mks = [_mono(a) for a in accs]

        def cnt_ge_groups(cand):
            c = jnp.where(mks[0] >= cand, onev, zero)
            for mk in mks[1:]:
                c = c + jnp.where(mk >= cand, onev, zero)
            return spl_i(jnp.sum(c))

        c0 = cnt_ge_groups(zero)
        t = jnp.where(c0 >= kvec, zero, tmin)

        def p2(i, t):
            cand = t | jnp.left_shift(onev, spl_i(jnp.int32(30) - i))
            c = cnt_ge_groups(cand)
            return jnp.where(c >= kvec, cand, t)
        t0 = lax.fori_loop(0, 31, p2, t)
        t0f = _key_to_float(t0)

        def p3(jj, cnt_l):
            vs, ms, offs = [], [], []
            c = cnt_l
            for g in range(8):
                v = rows_v[pl.ds(roff + (jj * 8 + g) * 16, 16)]
                m = v >= t0f
                vs.append(v)
                ms.append(m)
                offs.append(c)
                c = c + jnp.where(m, onev, zero)
            for g in range(8):
                plsc.store_scatter(cand_v, [lane_base + offs[g]], vs[g],
                                   mask=ms[g])
            return c
        cnt_l = lax.fori_loop(0, NVREG // 8, p3, zero)
        nv = jnp.max(cnt_l)
        nv4 = (nv + 3) // 4

        def cnt_ge_cands(candf):
            def inner(j4, acc):
                for u in range(4):
                    jj = j4 * 4 + u
                    cv = plsc.load_gather(cand_v, [lane_base + jj])
                    ok = (spl_i(jj) < cnt_l) & (cv >= candf)
                    acc = acc + jnp.where(ok, onev, zero)
                return acc
            accv = lax.fori_loop(0, nv4, inner, zero)
            return spl_i(jnp.sum(accv))

        c0e = cnt_ge_cands(jnp.zeros((16,), jnp.float32))
        te = jnp.where(c0e >= kvec, zero, tmin)

        def p4(i, t):
            cand = t | jnp.left_shift(onev, spl_i(jnp.int32(30) - i))
            c = cnt_ge_cands(_key_to_float(cand))
            return jnp.where(c >= kvec, cand, t)
        t1 = lax.fori_loop(0, 31, p4, te)

        plsc.store_scatter(th_v, [spl_i(rl)], _key_to_float(t1), mask=lane0)
        return carry

    lax.fori_loop(0, RPW, row_body, jnp.int32(0))
    pltpu.sync_copy(th_v, out_hbm.at[pl.ds(base_row, RPW)])


def _sc_thresholds(logits2d):
    mesh = plsc.VectorSubcoreMesh(core_axis_name="c", subcore_axis_name="s")
    return pl.kernel(
        _sc_body,
        mesh=mesh,
        out_type=jax.ShapeDtypeStruct((ROWS,), jnp.float32),
        scratch_types=[
            pltpu.VMEM((2 * N_LAT,), jnp.float32),
            pltpu.VMEM((N_LAT + 64,), jnp.float32),
            pltpu.VMEM((RPW,), jnp.float32),
            pltpu.SemaphoreType.DMA,
        ],
        compiler_params=pltpu.CompilerParams(needs_layout_passes=False),
    )(logits2d)


def _adec_body(lg_ref, th_ref, wd_ref, b_ref, alpha_ref, mask_ref, xhat_ref):
    k = pl.program_id(1)
    v = lg_ref[...]
    sel = v >= th_ref[...]
    alpha = jnp.where(sel, v, 0.0)
    alpha_ref[...] = alpha
    mask_ref[...] = sel & (v != 0.0)

    @pl.when(k == 0)
    def _init():
        xhat_ref[...] = jnp.broadcast_to(b_ref[...], (B3, N_IN))

    xhat_ref[...] += jax.lax.dot_general(
        alpha, wd_ref[...], (((1,), (1,)), ((), ())),
        preferred_element_type=jnp.float32,
    )


@jax.jit
def kernel(x, b, W_enc, b_enc, W_dec, miss_counts):
    del miss_counts
    b2 = b.reshape(1, N_IN)
    benc2 = b_enc.reshape(1, N_LAT)

    logits = pl.pallas_call(
        _enc_body,
        grid=(N_LAT // L1, ROWS // B1),
        in_specs=[
            pl.BlockSpec((B1, N_IN), lambda l, r: (r, 0)),
            pl.BlockSpec((1, N_IN), lambda l, r: (0, 0)),
            pl.BlockSpec((L1, N_IN), lambda l, r: (l, 0)),
            pl.BlockSpec((1, L1), lambda l, r: (0, l)),
        ],
        out_specs=pl.BlockSpec((B1, L1), lambda l, r: (r, l)),
        out_shape=jax.ShapeDtypeStruct((ROWS, N_LAT), jnp.float32),
        compiler_params=pltpu.CompilerParams(
            dimension_semantics=("arbitrary", "arbitrary"),
        ),
    )(x, b2, W_enc, benc2)

    thresh = _sc_thresholds(logits).reshape(ROWS, 1)

    alpha, mask, xhat = pl.pallas_call(
        _adec_body,
        grid=(ROWS // B3, N_LAT // L3),
        in_specs=[
            pl.BlockSpec((B3, L3), lambda r, k: (r, k)),
            pl.BlockSpec((B3, 1), lambda r, k: (r, 0)),
            pl.BlockSpec((N_IN, L3), lambda r, k: (0, k)),
            pl.BlockSpec((1, N_IN), lambda r, k: (0, 0)),
        ],
        out_specs=[
            pl.BlockSpec((B3, L3), lambda r, k: (r, k)),
            pl.BlockSpec((B3, L3), lambda r, k: (r, k)),
            pl.BlockSpec((B3, N_IN), lambda r, k: (r, 0)),
        ],
        out_shape=[
            jax.ShapeDtypeStruct((ROWS, N_LAT), jnp.float32),
            jax.ShapeDtypeStruct((ROWS, N_LAT), jnp.bool_),
            jax.ShapeDtypeStruct((ROWS, N_IN), jnp.float32),
        ],
        compiler_params=pltpu.CompilerParams(
            dimension_semantics=("arbitrary", "arbitrary"),
        ),
    )(logits, thresh, W_dec, b2)

    return (xhat, alpha, mask)
